# baseline (device time: 83368 ns/iter reference)
import jax
import jax.numpy as jnp
from jax import lax
from jax.experimental import pallas as pl
from jax.experimental.pallas import tpu as pltpu

N_DEV = 16


def kernel(x, Wq, K_ext, V_ext, Wo):
    B, Sq, Din = x.shape
    _, skv_per, Hq, Dh = K_ext.shape
    Dout = Wo.shape[1]
    Skv = N_DEV * skv_per
    HD = Hq * Dh

    def body(x_ref, wq_ref, k_ref, v_ref, wo_ref, out_ref,
             kvg_ref, send_sems, recv_sems):
        my = lax.axis_index("i")
        left = lax.rem(my + N_DEV - 1, N_DEV)
        right = lax.rem(my + 1, N_DEV)

        barrier = pltpu.get_barrier_semaphore()
        for nbr in (left, right):
            pl.semaphore_signal(barrier, inc=1, device_id=(nbr,),
                                device_id_type=pl.DeviceIdType.MESH)
        pl.semaphore_wait(barrier, 2)

        k = k_ref[:].reshape(B, skv_per, HD).astype(jnp.bfloat16)
        v = v_ref[:].reshape(B, skv_per, HD).astype(jnp.bfloat16)
        kvg_ref[pl.ds(my, 1)] = jnp.concatenate([k, v], axis=-1)[None]

        for h in range(N_DEV - 1):
            origin = lax.rem(my - h + N_DEV, N_DEV)
            rdma = pltpu.make_async_remote_copy(
                src_ref=kvg_ref.at[origin],
                dst_ref=kvg_ref.at[origin],
                send_sem=send_sems.at[h],
                recv_sem=recv_sems.at[h],
                device_id=(right,),
                device_id_type=pl.DeviceIdType.MESH,
            )
            rdma.start()
            rdma.wait()

        xb = x_ref[:].astype(jnp.bfloat16).reshape(B * Sq, Din)
        wq = wq_ref[:].astype(jnp.bfloat16)
        q = jnp.dot(xb, wq, preferred_element_type=jnp.float32)
        q = (q * 0.125).astype(jnp.bfloat16).reshape(B, Sq, Hq, Dh)

        g = kvg_ref[:]

        qb = lax.broadcasted_iota(jnp.int32, (Sq, Skv), 0) // 64
        kb = lax.broadcasted_iota(jnp.int32, (Sq, Skv), 1) // 64
        mask = (qb == kb) | ((kb % 4) == (qb % 4))

        ctx_parts = []
        for b in range(B):
            kv_b = g[:, b].reshape(Skv, 2 * HD)
            heads = []
            for h in range(Hq):
                k_bh = kv_b[:, h * Dh:(h + 1) * Dh]
                v_bh = kv_b[:, HD + h * Dh:HD + (h + 1) * Dh]
                s = lax.dot_general(
                    q[b, :, h, :], k_bh, (((1,), (1,)), ((), ())),
                    preferred_element_type=jnp.float32)
                s = jnp.where(mask, s, -1e9)
                m = jnp.max(s, axis=-1, keepdims=True)
                w = jnp.exp(s - m)
                w = w / jnp.sum(w, axis=-1, keepdims=True)
                heads.append(jnp.dot(w.astype(jnp.bfloat16), v_bh,
                                     preferred_element_type=jnp.float32))
            ctx_parts.append(jnp.concatenate(heads, axis=-1))
        ctx = jnp.stack(ctx_parts).reshape(B * Sq, HD)
        out = jnp.dot(ctx.astype(jnp.bfloat16), wo_ref[:].astype(jnp.bfloat16),
                      preferred_element_type=jnp.float32)
        out_ref[:] = out.reshape(B, Sq, Dout)

    return pl.pallas_call(
        body,
        out_shape=jax.ShapeDtypeStruct((B, Sq, Dout), jnp.float32),
        in_specs=[pl.BlockSpec(memory_space=pltpu.VMEM)] * 5,
        out_specs=pl.BlockSpec(memory_space=pltpu.VMEM),
        scratch_shapes=[
            pltpu.VMEM((N_DEV, B, skv_per, 2 * HD), jnp.bfloat16),
            pltpu.SemaphoreType.DMA((N_DEV - 1,)),
            pltpu.SemaphoreType.DMA((N_DEV - 1,)),
        ],
        compiler_params=pltpu.CompilerParams(collective_id=0),
    )(x, Wq, K_ext, V_ext, Wo)


# device time: 44104 ns/iter; 1.8903x vs baseline; 1.8903x over previous
import jax
import jax.numpy as jnp
from jax import lax
from jax.experimental import pallas as pl
from jax.experimental.pallas import tpu as pltpu

N_DEV = 16
N_STEPS = 4


def kernel(x, Wq, K_ext, V_ext, Wo):
    B, Sq, Din = x.shape
    _, skv_per, Hq, Dh = K_ext.shape
    Dout = Wo.shape[1]
    HD = Hq * Dh
    BS = B * Sq

    def body(x_ref, wq_ref, k_ref, v_ref, wo_ref, out_ref,
             o_acc, ms_acc, o_rcv, ms_rcv,
             o_ssem, o_rsem, ms_ssem, ms_rsem):
        my = lax.axis_index("i")

        barrier = pltpu.get_barrier_semaphore()
        for s in range(N_STEPS):
            pl.semaphore_signal(barrier, inc=1, device_id=(my ^ (1 << s),),
                                device_id_type=pl.DeviceIdType.MESH)
        pl.semaphore_wait(barrier, N_STEPS)

        xb = x_ref[:].astype(jnp.bfloat16).reshape(BS, Din)
        q = jnp.dot(xb, wq_ref[:].astype(jnp.bfloat16),
                    preferred_element_type=jnp.float32)
        q = (q * 0.125).astype(jnp.bfloat16).reshape(B, Sq, Hq, Dh)

        qb = lax.broadcasted_iota(jnp.int32, (Sq, skv_per), 0) // 64
        kb = my * (skv_per // 64) + \
            lax.broadcasted_iota(jnp.int32, (Sq, skv_per), 1) // 64
        mask = (qb == kb) | ((kb % 4) == (qb % 4))

        o_rows, m_rows, s_rows = [], [], []
        for b in range(B):
            heads_o, heads_m, heads_s = [], [], []
            for h in range(Hq):
                k_bh = k_ref[b, :, h, :].astype(jnp.bfloat16)
                v_bh = v_ref[b, :, h, :].astype(jnp.bfloat16)
                sc = lax.dot_general(
                    q[b, :, h, :], k_bh, (((1,), (1,)), ((), ())),
                    preferred_element_type=jnp.float32)
                sc = jnp.where(mask, sc, -1e9)
                m_bh = jnp.max(sc, axis=-1)
                w = jnp.where(mask, jnp.exp(sc - m_bh[:, None]), 0.0)
                heads_s.append(jnp.sum(w, axis=-1))
                heads_m.append(m_bh)
                heads_o.append(jnp.dot(w.astype(jnp.bfloat16), v_bh,
                                       preferred_element_type=jnp.float32))
            o_rows.append(jnp.concatenate(heads_o, axis=-1))
            m_rows.append(jnp.stack(heads_m, axis=-1))
            s_rows.append(jnp.stack(heads_s, axis=-1))

        o_acc[:] = jnp.concatenate(o_rows, axis=0)
        ms_acc[:, 0:Hq] = jnp.concatenate(m_rows, axis=0)
        ms_acc[:, Hq:] = jnp.concatenate(s_rows, axis=0)

        for step in range(N_STEPS):
            partner = my ^ (1 << step)
            r_o = pltpu.make_async_remote_copy(
                src_ref=o_acc, dst_ref=o_rcv.at[step],
                send_sem=o_ssem.at[step], recv_sem=o_rsem.at[step],
                device_id=(partner,), device_id_type=pl.DeviceIdType.MESH,
            )
            r_ms = pltpu.make_async_remote_copy(
                src_ref=ms_acc, dst_ref=ms_rcv.at[step],
                send_sem=ms_ssem.at[step], recv_sem=ms_rsem.at[step],
                device_id=(partner,), device_id_type=pl.DeviceIdType.MESH,
            )
            r_o.start()
            r_ms.start()
            r_o.wait()
            r_ms.wait()

            o_a = o_acc[:].reshape(BS, Hq, Dh)
            o_p = o_rcv[step].reshape(BS, Hq, Dh)
            m_a = ms_acc[:, 0:Hq]
            s_a = ms_acc[:, Hq:]
            m_p = ms_rcv[step, :, 0:Hq]
            s_p = ms_rcv[step, :, Hq:]

            m_n = jnp.maximum(m_a, m_p)
            alpha = jnp.exp(m_a - m_n)
            beta = jnp.exp(m_p - m_n)
            o_acc[:] = (o_a * alpha[:, :, None]
                        + o_p * beta[:, :, None]).reshape(BS, HD)
            ms_acc[:, 0:Hq] = m_n
            ms_acc[:, Hq:] = s_a * alpha + s_p * beta

        o = o_acc[:].reshape(BS, Hq, Dh)
        s_all = ms_acc[:, Hq:]
        ctx = (o / s_all[:, :, None]).reshape(BS, HD)
        out = jnp.dot(ctx.astype(jnp.bfloat16), wo_ref[:].astype(jnp.bfloat16),
                      preferred_element_type=jnp.float32)
        out_ref[:] = out.reshape(B, Sq, Dout)

    return pl.pallas_call(
        body,
        out_shape=jax.ShapeDtypeStruct((B, Sq, Dout), jnp.float32),
        in_specs=[pl.BlockSpec(memory_space=pltpu.VMEM)] * 5,
        out_specs=pl.BlockSpec(memory_space=pltpu.VMEM),
        scratch_shapes=[
            pltpu.VMEM((BS, HD), jnp.float32),
            pltpu.VMEM((BS, 2 * Hq), jnp.float32),
            pltpu.VMEM((N_STEPS, BS, HD), jnp.float32),
            pltpu.VMEM((N_STEPS, BS, 2 * Hq), jnp.float32),
            pltpu.SemaphoreType.DMA((N_STEPS,)),
            pltpu.SemaphoreType.DMA((N_STEPS,)),
            pltpu.SemaphoreType.DMA((N_STEPS,)),
            pltpu.SemaphoreType.DMA((N_STEPS,)),
        ],
        compiler_params=pltpu.CompilerParams(collective_id=0),
    )(x, Wq, K_ext, V_ext, Wo)


# device time: 34560 ns/iter; 2.4123x vs baseline; 1.2762x over previous
import jax
import jax.numpy as jnp
from jax import lax
from jax.experimental import pallas as pl
from jax.experimental.pallas import tpu as pltpu

N_DEV = 16
N_STEPS = 4


def kernel(x, Wq, K_ext, V_ext, Wo):
    B, Sq, Din = x.shape
    _, skv_per, Hq, Dh = K_ext.shape
    Dout = Wo.shape[1]
    HD = Hq * Dh
    BS = B * Sq

    def body(x_ref, wq_ref, k_ref, v_ref, wo_ref, out_ref,
             o_acc, o_snd, ms_acc, o_rcv, ms_rcv,
             o_ssem, o_rsem, ms_ssem, ms_rsem):
        my = lax.axis_index("i")

        xb = x_ref[:].astype(jnp.bfloat16).reshape(BS, Din)
        q = jnp.dot(xb, wq_ref[:].astype(jnp.bfloat16),
                    preferred_element_type=jnp.float32)
        q = (q * 0.125).astype(jnp.bfloat16).reshape(B, Sq, Hq, Dh)

        qb = lax.broadcasted_iota(jnp.int32, (Sq, skv_per), 0) // 64
        kb = my * (skv_per // 64) + \
            lax.broadcasted_iota(jnp.int32, (Sq, skv_per), 1) // 64
        mask = (qb == kb) | ((kb % 4) == (qb % 4))

        o_rows, m_rows, s_rows = [], [], []
        for b in range(B):
            heads_o, heads_m, heads_s = [], [], []
            for h in range(Hq):
                k_bh = k_ref[b, :, h, :].astype(jnp.bfloat16)
                v_bh = v_ref[b, :, h, :].astype(jnp.bfloat16)
                sc = lax.dot_general(
                    q[b, :, h, :], k_bh, (((1,), (1,)), ((), ())),
                    preferred_element_type=jnp.float32)
                sc = jnp.where(mask, sc, -1e9)
                m_bh = jnp.max(sc, axis=-1)
                w = jnp.where(mask, jnp.exp(sc - m_bh[:, None]), 0.0)
                heads_s.append(jnp.sum(w, axis=-1))
                heads_m.append(m_bh)
                heads_o.append(jnp.dot(w.astype(jnp.bfloat16), v_bh,
                                       preferred_element_type=jnp.float32))
            o_rows.append(jnp.concatenate(heads_o, axis=-1))
            m_rows.append(jnp.stack(heads_m, axis=-1))
            s_rows.append(jnp.stack(heads_s, axis=-1))

        o_local = jnp.concatenate(o_rows, axis=0)
        o_acc[:] = o_local
        o_snd[:] = o_local.astype(jnp.bfloat16)
        ms_acc[:, 0:Hq] = jnp.concatenate(m_rows, axis=0)
        ms_acc[:, Hq:] = jnp.concatenate(s_rows, axis=0)

        barrier = pltpu.get_barrier_semaphore()
        for s in range(N_STEPS):
            pl.semaphore_signal(barrier, inc=1, device_id=(my ^ (1 << s),),
                                device_id_type=pl.DeviceIdType.MESH)
        pl.semaphore_wait(barrier, N_STEPS)

        for step in range(N_STEPS):
            partner = my ^ (1 << step)
            r_o = pltpu.make_async_remote_copy(
                src_ref=o_snd, dst_ref=o_rcv.at[step],
                send_sem=o_ssem.at[step], recv_sem=o_rsem.at[step],
                device_id=(partner,), device_id_type=pl.DeviceIdType.MESH,
            )
            r_ms = pltpu.make_async_remote_copy(
                src_ref=ms_acc, dst_ref=ms_rcv.at[step],
                send_sem=ms_ssem.at[step], recv_sem=ms_rsem.at[step],
                device_id=(partner,), device_id_type=pl.DeviceIdType.MESH,
            )
            r_o.start()
            r_ms.start()
            r_o.wait()
            r_ms.wait()

            m_a = ms_acc[:, 0:Hq]
            s_a = ms_acc[:, Hq:]
            m_p = ms_rcv[step, :, 0:Hq]
            s_p = ms_rcv[step, :, Hq:]
            m_n = jnp.maximum(m_a, m_p)
            alpha = jnp.exp(m_a - m_n)
            beta = jnp.exp(m_p - m_n)
            for h in range(Hq):
                sl = slice(h * Dh, (h + 1) * Dh)
                o_h = (o_acc[:, sl] * alpha[:, h:h + 1]
                       + o_rcv[step, :, sl].astype(jnp.float32)
                       * beta[:, h:h + 1])
                o_acc[:, sl] = o_h
                if step < N_STEPS - 1:
                    o_snd[:, sl] = o_h.astype(jnp.bfloat16)
            ms_acc[:, 0:Hq] = m_n
            ms_acc[:, Hq:] = s_a * alpha + s_p * beta

        s_all = ms_acc[:, Hq:]
        ctx_cols = []
        for h in range(Hq):
            sl = slice(h * Dh, (h + 1) * Dh)
            ctx_cols.append((o_acc[:, sl] / s_all[:, h:h + 1])
                            .astype(jnp.bfloat16))
        ctx = jnp.concatenate(ctx_cols, axis=-1)
        out = jnp.dot(ctx, wo_ref[:].astype(jnp.bfloat16),
                      preferred_element_type=jnp.float32)
        out_ref[:] = out.reshape(B, Sq, Dout)

    return pl.pallas_call(
        body,
        out_shape=jax.ShapeDtypeStruct((B, Sq, Dout), jnp.float32),
        in_specs=[pl.BlockSpec(memory_space=pltpu.VMEM)] * 5,
        out_specs=pl.BlockSpec(memory_space=pltpu.VMEM),
        scratch_shapes=[
            pltpu.VMEM((BS, HD), jnp.float32),
            pltpu.VMEM((BS, HD), jnp.bfloat16),
            pltpu.VMEM((BS, 2 * Hq), jnp.float32),
            pltpu.VMEM((N_STEPS, BS, HD), jnp.bfloat16),
            pltpu.VMEM((N_STEPS, BS, 2 * Hq), jnp.float32),
            pltpu.SemaphoreType.DMA((N_STEPS,)),
            pltpu.SemaphoreType.DMA((N_STEPS,)),
            pltpu.SemaphoreType.DMA((N_STEPS,)),
            pltpu.SemaphoreType.DMA((N_STEPS,)),
        ],
        compiler_params=pltpu.CompilerParams(collective_id=0),
    )(x, Wq, K_ext, V_ext, Wo)


# device time: 31447 ns/iter; 2.6511x vs baseline; 1.0990x over previous
import jax
import jax.numpy as jnp
from jax import lax
from jax.experimental import pallas as pl
from jax.experimental.pallas import tpu as pltpu

N_DEV = 16
ROUND_OFFS = ((1, 2, 3), (4, 8, 12))
N_SLOTS = 6


def kernel(x, Wq, K_ext, V_ext, Wo):
    B, Sq, Din = x.shape
    _, skv_per, Hq, Dh = K_ext.shape
    Dout = Wo.shape[1]
    HD = Hq * Dh
    BS = B * Sq

    def body(x_ref, wq_ref, k_ref, v_ref, wo_ref, out_ref,
             o_acc, o_snd, ms_acc, ms_snd, o_rcv, ms_rcv,
             o_ssem, o_rsem, ms_ssem, ms_rsem):
        my = lax.axis_index("i")

        xb = x_ref[:].astype(jnp.bfloat16).reshape(BS, Din)
        q = jnp.dot(xb, wq_ref[:].astype(jnp.bfloat16),
                    preferred_element_type=jnp.float32)
        q = (q * 0.125).astype(jnp.bfloat16).reshape(B, Sq, Hq, Dh)

        qb = lax.broadcasted_iota(jnp.int32, (Sq, skv_per), 0) // 64
        kb = my * (skv_per // 64) + \
            lax.broadcasted_iota(jnp.int32, (Sq, skv_per), 1) // 64
        mask = (qb == kb) | ((kb % 4) == (qb % 4))

        o_rows, m_rows, s_rows = [], [], []
        for b in range(B):
            heads_o, heads_m, heads_s = [], [], []
            for h in range(Hq):
                k_bh = k_ref[b, :, h, :].astype(jnp.bfloat16)
                v_bh = v_ref[b, :, h, :].astype(jnp.bfloat16)
                sc = lax.dot_general(
                    q[b, :, h, :], k_bh, (((1,), (1,)), ((), ())),
                    preferred_element_type=jnp.float32)
                sc = jnp.where(mask, sc, -1e9)
                m_bh = jnp.max(sc, axis=-1)
                w = jnp.where(mask, jnp.exp(sc - m_bh[:, None]), 0.0)
                heads_s.append(jnp.sum(w, axis=-1))
                heads_m.append(m_bh)
                heads_o.append(jnp.dot(w.astype(jnp.bfloat16), v_bh,
                                       preferred_element_type=jnp.float32))
            o_rows.append(jnp.concatenate(heads_o, axis=-1))
            m_rows.append(jnp.stack(heads_m, axis=-1))
            s_rows.append(jnp.stack(heads_s, axis=-1))

        o_local = jnp.concatenate(o_rows, axis=0)
        o_acc[:] = o_local
        o_snd[:] = o_local.astype(jnp.bfloat16)
        ms_acc[:, 0:Hq] = jnp.concatenate(m_rows, axis=0)
        ms_acc[:, Hq:] = jnp.concatenate(s_rows, axis=0)
        ms_snd[:] = ms_acc[:]

        barrier = pltpu.get_barrier_semaphore()
        for offs in ROUND_OFFS:
            for d in offs:
                pl.semaphore_signal(barrier, inc=1, device_id=(my ^ d,),
                                    device_id_type=pl.DeviceIdType.MESH)
        pl.semaphore_wait(barrier, N_SLOTS)

        for r, offs in enumerate(ROUND_OFFS):
            rdmas = []
            for j, d in enumerate(offs):
                slot = r * 3 + j
                partner = my ^ d
                r_o = pltpu.make_async_remote_copy(
                    src_ref=o_snd, dst_ref=o_rcv.at[slot],
                    send_sem=o_ssem.at[slot], recv_sem=o_rsem.at[slot],
                    device_id=(partner,), device_id_type=pl.DeviceIdType.MESH,
                )
                r_ms = pltpu.make_async_remote_copy(
                    src_ref=ms_snd, dst_ref=ms_rcv.at[slot],
                    send_sem=ms_ssem.at[slot], recv_sem=ms_rsem.at[slot],
                    device_id=(partner,), device_id_type=pl.DeviceIdType.MESH,
                )
                r_o.start()
                r_ms.start()
                rdmas.append((r_o, r_ms))
            for r_o, r_ms in rdmas:
                r_o.wait()
                r_ms.wait()

            m_list = [ms_acc[:, 0:Hq]] + \
                [ms_rcv[r * 3 + j, :, 0:Hq] for j in range(3)]
            s_list = [ms_acc[:, Hq:]] + \
                [ms_rcv[r * 3 + j, :, Hq:] for j in range(3)]
            m_n = m_list[0]
            for mm in m_list[1:]:
                m_n = jnp.maximum(m_n, mm)
            scales = [jnp.exp(mm - m_n) for mm in m_list]
            s_n = scales[0] * s_list[0]
            for sc_, ss in zip(scales[1:], s_list[1:]):
                s_n = s_n + sc_ * ss
            for h in range(Hq):
                sl = slice(h * Dh, (h + 1) * Dh)
                o_h = o_acc[:, sl] * scales[0][:, h:h + 1]
                for j in range(3):
                    o_h = o_h + (o_rcv[r * 3 + j, :, sl]
                                 .astype(jnp.float32)
                                 * scales[j + 1][:, h:h + 1])
                o_acc[:, sl] = o_h
                if r == 0:
                    o_snd[:, sl] = o_h.astype(jnp.bfloat16)
            ms_acc[:, 0:Hq] = m_n
            ms_acc[:, Hq:] = s_n
            if r == 0:
                ms_snd[:] = ms_acc[:]

        s_all = ms_acc[:, Hq:]
        ctx_cols = []
        for h in range(Hq):
            sl = slice(h * Dh, (h + 1) * Dh)
            ctx_cols.append((o_acc[:, sl] / s_all[:, h:h + 1])
                            .astype(jnp.bfloat16))
        ctx = jnp.concatenate(ctx_cols, axis=-1)
        out = jnp.dot(ctx, wo_ref[:].astype(jnp.bfloat16),
                      preferred_element_type=jnp.float32)
        out_ref[:] = out.reshape(B, Sq, Dout)

    return pl.pallas_call(
        body,
        out_shape=jax.ShapeDtypeStruct((B, Sq, Dout), jnp.float32),
        in_specs=[pl.BlockSpec(memory_space=pltpu.VMEM)] * 5,
        out_specs=pl.BlockSpec(memory_space=pltpu.VMEM),
        scratch_shapes=[
            pltpu.VMEM((BS, HD), jnp.float32),
            pltpu.VMEM((BS, HD), jnp.bfloat16),
            pltpu.VMEM((BS, 2 * Hq), jnp.float32),
            pltpu.VMEM((BS, 2 * Hq), jnp.float32),
            pltpu.VMEM((N_SLOTS, BS, HD), jnp.bfloat16),
            pltpu.VMEM((N_SLOTS, BS, 2 * Hq), jnp.float32),
            pltpu.SemaphoreType.DMA((N_SLOTS,)),
            pltpu.SemaphoreType.DMA((N_SLOTS,)),
            pltpu.SemaphoreType.DMA((N_SLOTS,)),
            pltpu.SemaphoreType.DMA((N_SLOTS,)),
        ],
        compiler_params=pltpu.CompilerParams(collective_id=0),
    )(x, Wq, K_ext, V_ext, Wo)
